# Initial kernel scaffold; baseline (speedup 1.0000x reference)
import jax
import jax.numpy as jnp
from jax.experimental import pallas as pl


def kernel(embeds, batch_edges, field, uncertainty, adj, samples_src, samples_tgt):
    raise NotImplementedError("write your pallas kernel here")


# speedup vs baseline: 10.7879x; 10.7879x over previous
import functools
import math

import jax
import jax.numpy as jnp
from jax import lax
from jax.experimental import pallas as pl
from jax.experimental.pallas import tpu as pltpu
from jax.experimental.pallas import tpu_sc as plsc

_H, _N, _D = 4, 10000, 128
_B, _S = 1024, 128
_SENT = 8
_NC, _NS = 2, 16
_NW = _NC * _NS
_BPW = _B // _NW
_L = 16
_NG = _S // _L
_NK = _D // _L


def _lanesum(x, lane_iota):
    for sh in (8, 4, 2, 1):
        x = x + x.at[lane_iota ^ sh].get(mode="promise_in_bounds")
    return x


def _sc_body(sidx, pos, aidx, uvec, emb, fld, adjf, out,
             aidx_v, anchor_v, field_v, sidx_v, rows_v, pos_v, lab_v,
             u_v, out_v):
    lane_iota = lax.iota(jnp.int32, _L)
    wid = lax.axis_index("s") * _NC + lax.axis_index("c")
    base = wid * _BPW
    pltpu.sync_copy(uvec, u_v)
    u16 = u_v[...]
    zeros = jnp.zeros((_L,), jnp.float32)

    def edge_body(j, pvec, eg):
        b = base + eg * _L + j
        pltpu.sync_copy(aidx.at[b], aidx_v)
        pltpu.sync_copy(emb.at[aidx_v], anchor_v)
        pltpu.sync_copy(fld.at[aidx_v], field_v)

        def head_body(h, softacc):
            z_vec = zeros
            n_vec = zeros
            for side in range(2):
                r = h * 2 + side
                pltpu.sync_copy(sidx.at[b, h, side], sidx_v)
                pltpu.sync_copy(emb.at[sidx_v], rows_v)
                pltpu.sync_copy(pos.at[b, h, side], pos_v)
                pltpu.sync_copy(adjf.at[pos_v], lab_v)

                a_vecs = [anchor_v[r, pl.ds(_L * k, _L)] for k in range(_NK)]
                f_vecs = [field_v[r, pl.ds(_L * k, _L)] for k in range(_NK)]

                def group_body(grp, carry):
                    z_c, n_c = carry

                    def samp_body(sj, sc):
                        d2v, dfv = sc
                        s = grp * _L + sj
                        d2 = zeros
                        df = zeros
                        for k in range(_NK):
                            g = rows_v[s, pl.ds(_L * k, _L)]
                            d = a_vecs[k] - g
                            d2 = d2 + d * d
                            df = df + d * f_vecs[k]
                        sd2 = _lanesum(d2, lane_iota)
                        sdf = _lanesum(df, lane_iota)
                        m = lane_iota == sj
                        d2v = jnp.where(m, sd2, d2v)
                        dfv = jnp.where(m, sdf, dfv)
                        return d2v, dfv

                    d2v, dfv = lax.fori_loop(0, _L, samp_body,
                                             (zeros, zeros))
                    y = jnp.full((_L,), 2.0 ** -6, jnp.float32)
                    for kk in range(-5, 7):
                        y = jnp.where(d2v >= 4.0 ** kk,
                                      jnp.float32(2.0 ** kk), y)
                    for _ in range(4):
                        y = 0.5 * (y + d2v / y)
                    dist = jnp.where(d2v > 0.0, y, 0.0)
                    e = jnp.exp(-dist)
                    labv = lab_v[pl.ds(grp * _L, _L)]
                    logit = dfv + u16 * labv
                    return z_c + e, n_c + logit * e

                z_vec, n_vec = lax.fori_loop(0, _NG, group_body,
                                             (z_vec, n_vec))
            z_tot = _lanesum(z_vec, lane_iota) + _SENT * math.exp(-1.0)
            n_tot = _lanesum(n_vec, lane_iota)
            return softacc + n_tot / z_tot

        softacc = lax.fori_loop(0, _H, head_body, zeros)
        pred = softacc * (1.0 / _H)
        return jnp.where(lane_iota == j, pred, pvec)

    for eg in range(_BPW // _L):
        pvec = lax.fori_loop(0, _L, functools.partial(edge_body, eg=eg),
                             zeros)
        sig = 1.0 / (1.0 + jnp.exp(-pvec))
        out_v[pl.ds(eg * _L, _L)] = sig
    pltpu.sync_copy(out_v, out.at[pl.ds(base, _BPW)])


@jax.jit
def kernel(embeds, batch_edges, field, uncertainty, adj, samples_src,
           samples_tgt):
    src = batch_edges[0, :]
    dst = batch_edges[1, :]
    hoff = (jnp.arange(_H, dtype=jnp.int32) * _N)[:, None, None]
    sidx = jnp.stack([samples_src + hoff, samples_tgt + hoff],
                     axis=2).transpose(1, 0, 2, 3)
    p_src = samples_src * _N + src[None, :, None]
    p_tgt = dst[None, :, None] * _N + samples_tgt
    pos = jnp.stack([p_src, p_tgt], axis=2).transpose(1, 0, 2, 3)
    nodes = jnp.stack([src, dst], axis=1)
    aidx = ((jnp.arange(_H, dtype=jnp.int32) * _N)[None, :, None]
            + nodes[:, None, :]).reshape(_B, 2 * _H)
    uvec = jnp.broadcast_to(uncertainty.reshape(-1)[:1], (_L,))

    emb = embeds.reshape(_H * _N, _D)
    fld = field.reshape(_H * _N, _D)
    adjf = adj.reshape(_N * _N)

    mesh = plsc.VectorSubcoreMesh(core_axis_name="c", subcore_axis_name="s")
    run = functools.partial(
        pl.kernel,
        out_type=jax.ShapeDtypeStruct((_B,), jnp.float32),
        mesh=mesh,
        scratch_types=[
            pltpu.VMEM((2 * _H,), jnp.int32),
            pltpu.VMEM((2 * _H, _D), jnp.float32),
            pltpu.VMEM((2 * _H, _D), jnp.float32),
            pltpu.VMEM((_S,), jnp.int32),
            pltpu.VMEM((_S, _D), jnp.float32),
            pltpu.VMEM((_S,), jnp.int32),
            pltpu.VMEM((_S,), jnp.float32),
            pltpu.VMEM((_L,), jnp.float32),
            pltpu.VMEM((_BPW,), jnp.float32),
        ],
    )(_sc_body)
    return run(sidx, pos, aidx, uvec, emb, fld, adjf)
